# baseline (speedup 1.0000x reference)
import functools

import jax
import jax.numpy as jnp
from jax import lax
from jax.experimental import pallas as pl
from jax.experimental.pallas import tpu as pltpu
from jax.experimental.pallas import tpu_sc as plsc

Z = 128
NUM_IN = Z * (Z + 1) // 2


def _softplus16(v):
    t = jnp.exp(-jnp.abs(v))
    u = t / (t + 2.0)
    u2 = u * u
    p = 1.0 + u2 * (
        1.0 / 3.0 + u2 * (1.0 / 5.0 + u2 * (1.0 / 7.0 + u2 * (1.0 / 9.0 + u2 * (1.0 / 11.0))))
    )
    return jnp.maximum(v, 0.0) + 2.0 * u * p


def kernel(x):
    B = x.shape[0]
    info = plsc.get_sparse_core_info()
    NC, NS = info.num_cores, info.num_subcores
    NW = NC * NS
    rows_per_w = B // NW
    mesh = plsc.VectorSubcoreMesh(core_axis_name="c", subcore_axis_name="s")

    @functools.partial(
        pl.kernel,
        out_type=jax.ShapeDtypeStruct((B, Z * Z), jnp.float32),
        mesh=mesh,
        scratch_types=[
            pltpu.VMEM((NUM_IN,), jnp.float32),
            pltpu.VMEM((Z * Z,), jnp.float32),
        ],
    )
    def run(x_hbm, out_hbm, x_v, l_v):
        wid = lax.axis_index("s") * NC + lax.axis_index("c")
        base = wid * rows_per_w
        zero16 = jnp.zeros((16,), jnp.float32)
        iota16 = lax.iota(jnp.int32, 16)

        def zero_body(i, _):
            l_v[pl.ds(i * 16, 16)] = zero16
            return 0

        lax.fori_loop(0, Z * Z // 16, zero_body, 0)

        def row_body(i, _):
            b = base + i
            pltpu.sync_copy(x_hbm.at[b], x_v)

            def r_body(r, _):
                off = r * (r + 1) // 2
                nfull = r // 16

                def j_body(j, _):
                    l_v[pl.ds(r * Z + j * 16, 16)] = x_v[pl.ds(off + j * 16, 16)]
                    return 0

                lax.fori_loop(0, nfull, j_body, 0)

                lane = iota16 + nfull * 16
                vals = x_v[pl.ds(off + nfull * 16, 16)]
                sp = _softplus16(vals)
                out = jnp.where(
                    lane < r, vals, jnp.where(lane == r, sp, jnp.zeros_like(vals))
                )
                l_v[pl.ds(r * Z + nfull * 16, 16)] = out
                return 0

            lax.fori_loop(0, Z, r_body, 0)
            pltpu.sync_copy(l_v, out_hbm.at[b])
            return 0

        lax.fori_loop(0, rows_per_w, row_body, 0)

    out = run(x)
    return out.reshape(B, Z, Z)


# speedup vs baseline: 1.4115x; 1.4115x over previous
import functools

import jax
import jax.numpy as jnp
from jax import lax
from jax.experimental import pallas as pl
from jax.experimental.pallas import tpu as pltpu
from jax.experimental.pallas import tpu_sc as plsc

Z = 128
NUM_IN = Z * (Z + 1) // 2
G = 2


def _softplus16(v):
    t = jnp.exp(-jnp.abs(v))
    u = t / (t + 2.0)
    u2 = u * u
    p = 1.0 + u2 * (
        1.0 / 3.0 + u2 * (1.0 / 5.0 + u2 * (1.0 / 7.0 + u2 * (1.0 / 9.0 + u2 * (1.0 / 11.0))))
    )
    return jnp.maximum(v, 0.0) + 2.0 * u * p


def kernel(x):
    B = x.shape[0]
    info = plsc.get_sparse_core_info()
    NC, NS = info.num_cores, info.num_subcores
    NW = NC * NS
    rows_per_w = B // NW
    n_pairs = rows_per_w // G
    mesh = plsc.VectorSubcoreMesh(core_axis_name="c", subcore_axis_name="s")

    @functools.partial(
        pl.kernel,
        out_type=jax.ShapeDtypeStruct((B * Z * Z,), jnp.float32),
        mesh=mesh,
        scratch_types=[
            pltpu.VMEM((G * NUM_IN,), jnp.float32),
            pltpu.VMEM((G * NUM_IN,), jnp.float32),
            pltpu.VMEM((G * Z * Z,), jnp.float32),
            pltpu.VMEM((G * Z * Z,), jnp.float32),
            pltpu.SemaphoreType.DMA,
            pltpu.SemaphoreType.DMA,
            pltpu.SemaphoreType.DMA,
            pltpu.SemaphoreType.DMA,
        ],
    )
    def run(x_hbm, out_hbm, x_v0, x_v1, l_v0, l_v1, in_sem0, in_sem1, out_sem0, out_sem1):
        x_vs = (x_v0, x_v1)
        l_vs = (l_v0, l_v1)
        in_sems = (in_sem0, in_sem1)
        out_sems = (out_sem0, out_sem1)
        wid = lax.axis_index("s") * NC + lax.axis_index("c")
        base = wid * rows_per_w
        zero16 = jnp.zeros((16,), jnp.float32)
        iota16 = lax.iota(jnp.int32, 16)

        def zero_body(i, _):
            for slot in range(2):
                l_vs[slot][pl.ds(i * 16, 16)] = zero16
            return 0

        lax.fori_loop(0, G * Z * Z // 16, zero_body, 0)

        def in_copy(pair, slot):
            return pltpu.make_async_copy(
                x_hbm.at[pl.ds((base + pair * G) * NUM_IN, G * NUM_IN)],
                x_vs[slot],
                in_sems[slot],
            )

        def out_copy(pair, slot):
            return pltpu.make_async_copy(
                l_vs[slot],
                out_hbm.at[pl.ds((base + pair * G) * Z * Z, G * Z * Z)],
                out_sems[slot],
            )

        in_copy(0, 0).start()
        in_copy(1, 1).start()

        def rebuild(slot, g):
            x_v = x_vs[slot]
            l_v = l_vs[slot]
            for k in range(Z // 16):
                def r_body(rr, _):
                    r = k * 16 + rr
                    off = g * NUM_IN + (r * (r + 1)) // 2
                    rowb = g * Z * Z + r * Z
                    for j in range(k):
                        l_v[pl.ds(rowb + j * 16, 16)] = x_v[pl.ds(off + j * 16, 16)]
                    lane = iota16 + k * 16
                    vals = x_v[pl.ds(off + k * 16, 16)]
                    sp = _softplus16(vals)
                    out = jnp.where(lane < r, vals, jnp.where(lane == r, sp, zero16))
                    l_v[pl.ds(rowb + k * 16, 16)] = out
                    return 0

                lax.fori_loop(0, 16, r_body, 0)

        def step(it, _):
            for slot in range(2):
                pair = it * 2 + slot
                in_copy(pair, slot).wait()

                @pl.when(it >= 1)
                def _():
                    out_copy(pair - 2, slot).wait()

                for g in range(G):
                    rebuild(slot, g)
                out_copy(pair, slot).start()

                @pl.when(pair + 2 < n_pairs)
                def _():
                    in_copy(pair + 2, slot).start()

            return 0

        lax.fori_loop(0, n_pairs // 2, step, 0)

        out_copy(n_pairs - 2, 0).wait()
        out_copy(n_pairs - 1, 1).wait()

    out = run(x.reshape(B * NUM_IN))
    return out.reshape(B, Z, Z)


# speedup vs baseline: 3.4742x; 2.4613x over previous
import functools

import jax
import jax.numpy as jnp
from jax import lax
from jax.experimental import pallas as pl
from jax.experimental.pallas import tpu as pltpu
from jax.experimental.pallas import tpu_sc as plsc

Z = 128
NUM_IN = Z * (Z + 1) // 2
G = 2


def _softplus16(v):
    t = jnp.exp(-jnp.abs(v))
    u = t / (t + 2.0)
    u2 = u * u
    p = 1.0 + u2 * (
        1.0 / 3.0 + u2 * (1.0 / 5.0 + u2 * (1.0 / 7.0 + u2 * (1.0 / 9.0 + u2 * (1.0 / 11.0))))
    )
    return jnp.maximum(v, 0.0) + 2.0 * u * p


def kernel(x):
    B = x.shape[0]
    info = plsc.get_sparse_core_info()
    NC, NS = info.num_cores, info.num_subcores
    NW = NC * NS
    rows_per_w = B // NW
    n_pairs = rows_per_w // G
    mesh = plsc.VectorSubcoreMesh(core_axis_name="c", subcore_axis_name="s")

    @functools.partial(
        pl.kernel,
        out_type=jax.ShapeDtypeStruct((B * Z * Z,), jnp.float32),
        mesh=mesh,
        compiler_params=pltpu.CompilerParams(needs_layout_passes=False),
        scratch_types=[
            pltpu.VMEM((G * NUM_IN,), jnp.float32),
            pltpu.VMEM((G * NUM_IN,), jnp.float32),
            pltpu.VMEM((G * Z * Z,), jnp.float32),
            pltpu.VMEM((G * Z * Z,), jnp.float32),
            pltpu.SemaphoreType.DMA,
            pltpu.SemaphoreType.DMA,
            pltpu.SemaphoreType.DMA,
            pltpu.SemaphoreType.DMA,
        ],
    )
    def run(x_hbm, out_hbm, x_v0, x_v1, l_v0, l_v1, in_sem0, in_sem1, out_sem0, out_sem1):
        x_vs = (x_v0, x_v1)
        l_vs = (l_v0, l_v1)
        in_sems = (in_sem0, in_sem1)
        out_sems = (out_sem0, out_sem1)
        wid = lax.axis_index("s") * NC + lax.axis_index("c")
        base = wid * rows_per_w
        zero16 = jnp.zeros((16,), jnp.float32)
        iota16 = lax.iota(jnp.int32, 16)

        @plsc.parallel_loop(0, G * Z * Z // 16, unroll=4)
        def _zero(i):
            for slot in range(2):
                l_vs[slot][pl.ds(i * 16, 16)] = zero16

        def in_copy(pair, slot):
            return pltpu.make_async_copy(
                x_hbm.at[pl.ds((base + pair * G) * NUM_IN, G * NUM_IN)],
                x_vs[slot],
                in_sems[slot],
            )

        def out_copy(pair, slot):
            return pltpu.make_async_copy(
                l_vs[slot],
                out_hbm.at[pl.ds((base + pair * G) * Z * Z, G * Z * Z)],
                out_sems[slot],
            )

        in_copy(0, 0).start()
        in_copy(1, 1).start()

        def rebuild(slot, g):
            x_v = x_vs[slot]
            l_v = l_vs[slot]
            for k in range(Z // 16):
                @plsc.parallel_loop(0, 16, unroll=2)
                def _row(rr):
                    r = k * 16 + rr
                    off = g * NUM_IN + (r * (r + 1)) // 2
                    rowb = g * Z * Z + r * Z
                    for j in range(k):
                        l_v[pl.ds(rowb + j * 16, 16)] = x_v[pl.ds(off + j * 16, 16)]
                    vals = x_v[pl.ds(off + k * 16, 16)]
                    l_v[pl.ds(rowb + k * 16, 16)] = jnp.where(iota16 < rr, vals, zero16)

            for k8 in range(Z // 16):
                r_vec = iota16 + k8 * 16
                src = lax.shift_right_logical(r_vec * (r_vec + 3), 1) + g * NUM_IN
                vals = plsc.load_gather(x_v, [src])
                sp = _softplus16(vals)
                dst = r_vec * (Z + 1) + g * Z * Z
                plsc.store_scatter(l_v, [dst], sp)

        def step(it, _):
            for slot in range(2):
                pair = it * 2 + slot
                in_copy(pair, slot).wait()

                @pl.when(it >= 1)
                def _():
                    out_copy(pair - 2, slot).wait()

                for g in range(G):
                    rebuild(slot, g)
                out_copy(pair, slot).start()

                @pl.when(pair + 2 < n_pairs)
                def _():
                    in_copy(pair + 2, slot).start()

            return 0

        lax.fori_loop(0, n_pairs // 2, step, 0)

        out_copy(n_pairs - 2, 0).wait()
        out_copy(n_pairs - 1, 1).wait()

    out = run(x.reshape(B * NUM_IN))
    return out.reshape(B, Z, Z)


# speedup vs baseline: 3.4845x; 1.0030x over previous
import functools

import jax
import jax.numpy as jnp
from jax import lax
from jax.experimental import pallas as pl
from jax.experimental.pallas import tpu as pltpu
from jax.experimental.pallas import tpu_sc as plsc

Z = 128
NUM_IN = Z * (Z + 1) // 2
NT = (NUM_IN + Z - 1) // Z
XW = NT * Z
HB = 4


def _softplus16(v):
    t = jnp.exp(-jnp.abs(v))
    u = t / (t + 2.0)
    u2 = u * u
    p = 1.0 + u2 * (
        1.0 / 3.0 + u2 * (1.0 / 5.0 + u2 * (1.0 / 7.0 + u2 * (1.0 / 9.0 + u2 * (1.0 / 11.0))))
    )
    return jnp.maximum(v, 0.0) + 2.0 * u * p


def kernel(x):
    B = x.shape[0]
    info = plsc.get_sparse_core_info()
    NC, NS = info.num_cores, info.num_subcores
    NW = NC * NS
    rows_per_w = B // NW
    n_hb = rows_per_w // HB
    mesh = plsc.VectorSubcoreMesh(core_axis_name="c", subcore_axis_name="s")

    @functools.partial(
        pl.kernel,
        out_type=jax.ShapeDtypeStruct((B * Z * Z,), jnp.float32),
        mesh=mesh,
        compiler_params=pltpu.CompilerParams(needs_layout_passes=False),
        scratch_types=[
            pltpu.VMEM((HB, XW), jnp.float32),
            pltpu.VMEM((HB, XW), jnp.float32),
            pltpu.VMEM((Z * Z,), jnp.float32),
            pltpu.VMEM((Z * Z,), jnp.float32),
            pltpu.SemaphoreType.DMA,
            pltpu.SemaphoreType.DMA,
            pltpu.SemaphoreType.DMA,
            pltpu.SemaphoreType.DMA,
        ],
    )
    def run(x_hbm, out_hbm, x_v0, x_v1, l_v0, l_v1, in_sem0, in_sem1, out_sem0, out_sem1):
        x_vs = (x_v0, x_v1)
        l_vs = (l_v0, l_v1)
        in_sems = (in_sem0, in_sem1)
        out_sems = (out_sem0, out_sem1)
        wid = lax.axis_index("s") * NC + lax.axis_index("c")
        base = wid * rows_per_w
        zero16 = jnp.zeros((16,), jnp.float32)
        iota16 = lax.iota(jnp.int32, 16)

        @plsc.parallel_loop(0, Z * Z // 16, unroll=4)
        def _zero(i):
            for slot in range(2):
                l_vs[slot][pl.ds(i * 16, 16)] = zero16

        def in_dma(hb, slot, t):
            row0 = base + hb * HB
            col = pl.multiple_of(t * Z, Z)
            return pltpu.make_async_copy(
                x_hbm.at[pl.ds(row0, HB), pl.ds(col, Z)],
                x_vs[slot].at[:, pl.ds(col, Z)],
                in_sems[slot],
            )

        def start_in(hb, slot):
            def t_body(t, _):
                in_dma(hb, slot, t).start()
                return 0

            lax.fori_loop(0, NT, t_body, 0)

        def wait_in(hb, slot):
            def t_body(t, _):
                in_dma(hb, slot, t).wait()
                return 0

            lax.fori_loop(0, NT, t_body, 0)

        def out_copy(m, lslot):
            return pltpu.make_async_copy(
                l_vs[lslot],
                out_hbm.at[pl.ds((base + m) * Z * Z, Z * Z)],
                out_sems[lslot],
            )

        start_in(0, 0)
        start_in(1, 1)

        def rebuild(x_v2, l_v, g):
            g_vec = jnp.full((16,), 0, jnp.int32) + g
            for k in range(Z // 16):
                @plsc.parallel_loop(0, 16, unroll=2)
                def _row(rr):
                    r = k * 16 + rr
                    off = (r * (r + 1)) // 2
                    rowb = r * Z
                    for j in range(k):
                        l_v[pl.ds(rowb + j * 16, 16)] = plsc.load_gather(
                            x_v2, [g_vec, off + j * 16 + iota16]
                        )
                    vals = plsc.load_gather(x_v2, [g_vec, off + k * 16 + iota16])
                    l_v[pl.ds(rowb + k * 16, 16)] = jnp.where(iota16 < rr, vals, zero16)

            for k8 in range(Z // 16):
                r_vec = iota16 + k8 * 16
                src = lax.shift_right_logical(r_vec * (r_vec + 3), 1)
                vals = plsc.load_gather(x_v2, [g_vec, src])
                sp = _softplus16(vals)
                plsc.store_scatter(l_v, [r_vec * (Z + 1)], sp)

        def step(it, _):
            for slot in range(2):
                hb = it * 2 + slot
                wait_in(hb, slot)

                def g_body(g2, _):
                    for lslot in range(2):
                        g = g2 * 2 + lslot
                        m = hb * HB + g

                        @pl.when(m >= 2)
                        def _():
                            out_copy(m - 2, lslot).wait()

                        rebuild(x_vs[slot], l_vs[lslot], g)
                        out_copy(m, lslot).start()
                    return 0

                lax.fori_loop(0, HB // 2, g_body, 0)

                @pl.when(hb + 2 < n_hb)
                def _():
                    start_in(hb + 2, slot)

            return 0

        lax.fori_loop(0, n_hb // 2, step, 0)

        out_copy(rows_per_w - 2, 0).wait()
        out_copy(rows_per_w - 1, 1).wait()

    out = run(x)
    return out.reshape(B, Z, Z)


# speedup vs baseline: 4.4027x; 1.2635x over previous
import functools

import jax
import jax.numpy as jnp
from jax import lax
from jax.experimental import pallas as pl
from jax.experimental.pallas import tpu as pltpu
from jax.experimental.pallas import tpu_sc as plsc

Z = 128
NUM_IN = Z * (Z + 1) // 2
NT = (NUM_IN + Z - 1) // Z
XW = NT * Z
HB = 4


def _softplus16(v):
    t = jnp.exp(-jnp.abs(v))
    u = t / (t + 2.0)
    u2 = u * u
    p = 1.0 + u2 * (
        1.0 / 3.0 + u2 * (1.0 / 5.0 + u2 * (1.0 / 7.0 + u2 * (1.0 / 9.0 + u2 * (1.0 / 11.0))))
    )
    return jnp.maximum(v, 0.0) + 2.0 * u * p


def kernel(x):
    B = x.shape[0]
    info = plsc.get_sparse_core_info()
    NC, NS = info.num_cores, info.num_subcores
    NW = NC * NS
    rows_per_w = B // NW
    n_hb = rows_per_w // HB
    mesh = plsc.VectorSubcoreMesh(core_axis_name="c", subcore_axis_name="s")

    @functools.partial(
        pl.kernel,
        out_type=jax.ShapeDtypeStruct((B * Z * Z,), jnp.float32),
        mesh=mesh,
        compiler_params=pltpu.CompilerParams(needs_layout_passes=False),
        scratch_types=[
            pltpu.VMEM((HB, XW), jnp.float32),
            pltpu.VMEM((HB, XW), jnp.float32),
            pltpu.VMEM((Z * Z,), jnp.float32),
            pltpu.VMEM((Z * Z,), jnp.float32),
            pltpu.SemaphoreType.DMA,
            pltpu.SemaphoreType.DMA,
            pltpu.SemaphoreType.DMA,
            pltpu.SemaphoreType.DMA,
        ],
    )
    def run(x_hbm, out_hbm, x_v0, x_v1, l_v0, l_v1, in_sem0, in_sem1, out_sem0, out_sem1):
        x_vs = (x_v0, x_v1)
        l_vs = (l_v0, l_v1)
        in_sems = (in_sem0, in_sem1)
        out_sems = (out_sem0, out_sem1)
        wid = lax.axis_index("s") * NC + lax.axis_index("c")
        base = wid * rows_per_w
        zero16 = jnp.zeros((16,), jnp.float32)
        iota16 = lax.iota(jnp.int32, 16)

        @plsc.parallel_loop(0, Z * Z // 16, unroll=4)
        def _zero(i):
            for slot in range(2):
                l_vs[slot][pl.ds(i * 16, 16)] = zero16

        def in_dma(hb, slot, t):
            row0 = base + hb * HB
            col = pl.multiple_of(t * Z, Z)
            return pltpu.make_async_copy(
                x_hbm.at[pl.ds(row0, HB), pl.ds(col, Z)],
                x_vs[slot].at[:, pl.ds(col, Z)],
                in_sems[slot],
            )

        def start_in(hb, slot):
            def t_body(t, _):
                in_dma(hb, slot, t).start()
                return 0

            lax.fori_loop(0, NT, t_body, 0)

        def wait_in(hb, slot):
            for g in range(HB):
                pltpu.make_async_copy(
                    out_hbm.at[pl.ds(0, XW)], x_vs[slot].at[g], in_sems[slot]
                ).wait()

        def out_copy(m, lslot):
            return pltpu.make_async_copy(
                l_vs[lslot],
                out_hbm.at[pl.ds((base + m) * Z * Z, Z * Z)],
                out_sems[lslot],
            )

        start_in(0, 0)
        start_in(1, 1)

        def rebuild(x_v2, l_v, g):
            g_vec = jnp.full((16,), g, jnp.int32)
            for j in range(Z // 16 - 1):
                @plsc.parallel_loop(16 * (j + 1), Z, unroll=2)
                def _c(r):
                    off = (r * (r + 1)) // 2
                    l_v[pl.ds(r * Z + j * 16, 16)] = plsc.load_gather(
                        x_v2, [g_vec, off + j * 16 + iota16]
                    )

            @plsc.parallel_loop(0, Z, unroll=2)
            def _b(r):
                k16 = jnp.bitwise_and(r, ~15)
                off = (r * (r + 1)) // 2
                vals = plsc.load_gather(x_v2, [g_vec, off + k16 + iota16])
                rr = jnp.bitwise_and(r, 15)
                l_v[pl.ds(r * Z + k16, 16)] = jnp.where(iota16 < rr, vals, zero16)

            @plsc.parallel_loop(0, Z // 16)
            def _d(k8):
                r_vec = iota16 + k8 * 16
                srcv = lax.shift_right_logical(r_vec * (r_vec + 3), 1)
                vals = plsc.load_gather(x_v2, [g_vec, srcv])
                sp = _softplus16(vals)
                plsc.store_scatter(l_v, [r_vec * (Z + 1)], sp)

        def step(it, _):
            for slot in range(2):
                hb = it * 2 + slot
                wait_in(hb, slot)

                for g in range(HB):
                    lslot = g % 2
                    m = hb * HB + g

                    @pl.when(m >= 2)
                    def _():
                        out_copy(m - 2, lslot).wait()

                    rebuild(x_vs[slot], l_vs[lslot], g)
                    out_copy(m, lslot).start()

                @pl.when(hb + 2 < n_hb)
                def _():
                    start_in(hb + 2, slot)

            return 0

        lax.fori_loop(0, n_hb // 2, step, 0)

        out_copy(rows_per_w - 2, 0).wait()
        out_copy(rows_per_w - 1, 1).wait()

    out = run(x)
    return out.reshape(B, Z, Z)


# speedup vs baseline: 4.8909x; 1.1109x over previous
import functools

import jax
import jax.numpy as jnp
from jax import lax
from jax.experimental import pallas as pl
from jax.experimental.pallas import tpu as pltpu
from jax.experimental.pallas import tpu_sc as plsc

Z = 128
NUM_IN = Z * (Z + 1) // 2
NT = (NUM_IN + Z - 1) // Z
XW = NT * Z
HB = 4


def _softplus16(v):
    t = jnp.exp(-jnp.abs(v))
    u = t / (t + 2.0)
    u2 = u * u
    p = 1.0 + u2 * (
        1.0 / 3.0 + u2 * (1.0 / 5.0 + u2 * (1.0 / 7.0 + u2 * (1.0 / 9.0 + u2 * (1.0 / 11.0))))
    )
    return jnp.maximum(v, 0.0) + 2.0 * u * p


def kernel(x):
    B = x.shape[0]
    info = plsc.get_sparse_core_info()
    NC, NS = info.num_cores, info.num_subcores
    NW = NC * NS
    rows_per_w = B // NW
    n_hb = rows_per_w // HB
    mesh = plsc.VectorSubcoreMesh(core_axis_name="c", subcore_axis_name="s")

    @functools.partial(
        pl.kernel,
        out_type=jax.ShapeDtypeStruct((B * Z * Z,), jnp.float32),
        mesh=mesh,
        compiler_params=pltpu.CompilerParams(needs_layout_passes=False),
        scratch_types=[
            pltpu.VMEM((HB, XW), jnp.float32),
            pltpu.VMEM((HB, XW), jnp.float32),
            pltpu.VMEM((Z * Z,), jnp.float32),
            pltpu.VMEM((Z * Z,), jnp.float32),
            pltpu.SemaphoreType.DMA,
            pltpu.SemaphoreType.DMA,
            pltpu.SemaphoreType.DMA,
            pltpu.SemaphoreType.DMA,
        ],
    )
    def run(x_hbm, out_hbm, x_v0, x_v1, l_v0, l_v1, in_sem0, in_sem1, out_sem0, out_sem1):
        x_vs = (x_v0, x_v1)
        l_vs = (l_v0, l_v1)
        in_sems = (in_sem0, in_sem1)
        out_sems = (out_sem0, out_sem1)
        wid = lax.axis_index("s") * NC + lax.axis_index("c")
        base = wid * rows_per_w
        zero16 = jnp.zeros((16,), jnp.float32)
        iota16 = lax.iota(jnp.int32, 16)

        @plsc.parallel_loop(0, Z * Z // 16, unroll=4)
        def _zero(i):
            for slot in range(2):
                l_vs[slot][pl.ds(i * 16, 16)] = zero16

        def in_dma(hb, slot, t):
            row0 = base + hb * HB
            col = pl.multiple_of(t * Z, Z)
            return pltpu.make_async_copy(
                x_hbm.at[pl.ds(row0, HB), pl.ds(col, Z)],
                x_vs[slot].at[:, pl.ds(col, Z)],
                in_sems[slot],
            )

        def start_in(hb, slot):
            def t_body(t, _):
                in_dma(hb, slot, t).start()
                return 0

            lax.fori_loop(0, NT, t_body, 0)

        def wait_in(hb, slot):
            for g in range(HB):
                pltpu.make_async_copy(
                    out_hbm.at[pl.ds(0, XW)], x_vs[slot].at[g], in_sems[slot]
                ).wait()

        def out_copy(m, lslot):
            return pltpu.make_async_copy(
                l_vs[lslot],
                out_hbm.at[pl.ds((base + m) * Z * Z, Z * Z)],
                out_sems[lslot],
            )

        start_in(0, 0)
        start_in(1, 1)

        def rebuild(x_v2, l_v, g):
            g_vec = jnp.full((16,), g, jnp.int32)
            for j in range(Z // 16 - 1):
                @plsc.parallel_loop(16 * (j + 1), Z, unroll=4)
                def _c(r):
                    off = (r * (r + 1)) // 2
                    l_v[pl.ds(r * Z + j * 16, 16)] = plsc.load_gather(
                        x_v2, [g_vec, off + j * 16 + iota16]
                    )

            @plsc.parallel_loop(0, Z, unroll=4)
            def _b(r):
                k16 = jnp.bitwise_and(r, ~15)
                off = (r * (r + 1)) // 2
                vals = plsc.load_gather(x_v2, [g_vec, off + k16 + iota16])
                rr = jnp.bitwise_and(r, 15)
                l_v[pl.ds(r * Z + k16, 16)] = jnp.where(iota16 < rr, vals, zero16)

            @plsc.parallel_loop(0, Z // 16)
            def _d(k8):
                r_vec = iota16 + k8 * 16
                srcv = lax.shift_right_logical(r_vec * (r_vec + 3), 1)
                vals = plsc.load_gather(x_v2, [g_vec, srcv])
                sp = _softplus16(vals)
                plsc.store_scatter(l_v, [r_vec * (Z + 1)], sp)

        def step(it, _):
            for slot in range(2):
                hb = it * 2 + slot
                wait_in(hb, slot)

                for g in range(HB):
                    lslot = g % 2
                    m = hb * HB + g

                    @pl.when(m >= 2)
                    def _():
                        out_copy(m - 2, lslot).wait()

                    rebuild(x_vs[slot], l_vs[lslot], g)
                    out_copy(m, lslot).start()

                @pl.when(hb + 2 < n_hb)
                def _():
                    start_in(hb + 2, slot)

            return 0

        lax.fori_loop(0, n_hb // 2, step, 0)

        out_copy(rows_per_w - 2, 0).wait()
        out_copy(rows_per_w - 1, 1).wait()

    out = run(x)
    return out.reshape(B, Z, Z)
